# baseline (speedup 1.0000x reference)
import functools

import jax
import jax.numpy as jnp
from jax import lax
from jax.experimental import pallas as pl
from jax.experimental.pallas import tpu as pltpu
from jax.experimental.pallas import tpu_sc as plsc

_NC = 2
_NS = 16
_NT = _NC * _NS
_CH = 128
_PAD = 64


def _make_edge_agg(n, npad, nchunk, d, stage_x, nb_rows, nb_idx):
    mesh = plsc.VectorSubcoreMesh(core_axis_name="c", subcore_axis_name="s")
    nblk = n // 80
    tc_tiled = d % 128 == 0

    scratch = (
        [pltpu.VMEM((_CH,), jnp.int32) for _ in range(nb_idx)]
        + [pltpu.VMEM((_CH,), jnp.int32) for _ in range(nb_idx)]
        + [pltpu.VMEM((_CH, d), jnp.float32) for _ in range(nb_rows)]
        + [pltpu.VMEM_SHARED((n, d), jnp.float32)]
        + ([pltpu.VMEM_SHARED((npad, d), jnp.float32)] if stage_x else [])
        + [pltpu.SemaphoreType.DMA] * (2 * nb_idx + 2 * nb_rows)
    )

    @functools.partial(
        pl.kernel,
        out_type=jax.ShapeDtypeStruct((_NC, n, d), jnp.float32),
        mesh=mesh,
        compiler_params=pltpu.CompilerParams(use_tc_tiling_on_sc=(d % 128 == 0)),
        scratch_types=scratch,
    )
    def agg(*args):
        if tc_tiled:
            x_hbm, src_hbm, dst_hbm, zero_hbm, out_hbm, *refs = args
        else:
            x_hbm, src_hbm, dst_hbm, out_hbm, *refs = args
        sidx = refs[0:nb_idx]
        didx = refs[nb_idx:2 * nb_idx]
        rows = refs[2 * nb_idx:2 * nb_idx + nb_rows]
        agg_sh = refs[2 * nb_idx + nb_rows]
        x_src = refs[2 * nb_idx + nb_rows + 1] if stage_x else x_hbm
        sems = refs[2 * nb_idx + nb_rows + 1 + (1 if stage_x else 0):]
        ssem = sems[0:nb_idx]
        dsem = sems[nb_idx:2 * nb_idx]
        gsem = sems[2 * nb_idx:2 * nb_idx + nb_rows]
        wsem = sems[2 * nb_idx + nb_rows:]

        c = lax.axis_index("c")
        s = lax.axis_index("s")
        wid = c * _NS + s

        base = pl.multiple_of(wid * (nchunk * _CH), _CH)

        def iload(i):
            b = i % nb_idx
            sl = pl.ds(base + i * _CH, _CH)
            return (pltpu.async_copy(src_hbm.at[sl], sidx[b], ssem[b]),
                    pltpu.async_copy(dst_hbm.at[sl], didx[b], dsem[b]))

        def gather(i):
            b = i % nb_rows
            return pltpu.async_copy(x_src.at[sidx[i % nb_idx]], rows[b],
                                    gsem[b])

        def scat(i):
            b = i % nb_rows
            return pltpu.async_copy(rows[b], agg_sh.at[didx[i % nb_idx]],
                                    wsem[b], add=True)

        idd = {i: iload(i) for i in range(min(3, nchunk))}

        if tc_tiled:
            @pl.when(s == 0)
            def _():
                pltpu.sync_copy(zero_hbm, agg_sh)
        else:
            def zrow(r, carry):
                for cc in range(d // 16):
                    rows[0][r, pl.ds(cc * 16, 16)] = jnp.zeros(
                        (16,), jnp.float32)
                return carry
            lax.fori_loop(0, 80, zrow, 0)
            for j in range(-(-nblk // _NS)):
                blk = j * _NS + s

                @pl.when(blk < nblk)
                def _():
                    pltpu.sync_copy(rows[0].at[pl.ds(0, 80)],
                                    agg_sh.at[pl.ds(blk * 80, 80)])

        if stage_x:
            @pl.when(s == 0)
            def _():
                pltpu.sync_copy(x_hbm, x_src)

        plsc.subcore_barrier()

        gd = {}
        for i in range(min(2, nchunk)):
            idd[i][0].wait()
            gd[i] = gather(i)

        sd = {}
        for i in range(nchunk):
            gd[i].wait()
            idd[i][1].wait()
            sd[i] = scat(i)
            if i + 3 < nchunk:
                if i >= 1:
                    sd[i - 1].wait()
                idd[i + 3] = iload(i + 3)
            if i + 2 < nchunk:
                idd[i + 2][0].wait()
                gd[i + 2] = gather(i + 2)
        for i in range(max(0, nchunk - 4), nchunk):
            sd[i].wait()

        plsc.subcore_barrier()

        for cc in range(_NC):
            @pl.when((s == 0) & (c == cc))
            def _():
                pltpu.sync_copy(agg_sh, out_hbm.at[cc])

    return agg


def _lrelu(x):
    return jnp.where(x >= 0, x, 0.01 * x)


def _bn(y, g, b):
    m = jnp.mean(y, axis=0, keepdims=True)
    v = jnp.mean((y - m) * (y - m), axis=0, keepdims=True)
    return g * (y - m) * lax.rsqrt(v + 1e-5) + b


def _make_ctrb(n):
    def body(idx_ref, ed_ref, el_ref, wf1_ref, bf1_ref, out_ref):
        d = 128
        it = lax.broadcasted_iota(jnp.int32, (n, 128), 1)
        ohd = (idx_ref[:, 0:1] == it).astype(jnp.float32)
        ohl = (idx_ref[:, 1:2] == it).astype(jnp.float32)
        ed_p = jnp.concatenate(
            [ed_ref[...], jnp.zeros((128 - ed_ref.shape[0], d), jnp.float32)],
            axis=0)
        el_p = jnp.concatenate(
            [el_ref[...], jnp.zeros((128 - el_ref.shape[0], d), jnp.float32)],
            axis=0)
        edw = jnp.dot(ed_p, wf1_ref[0:d], preferred_element_type=jnp.float32)
        elw = jnp.dot(el_p, wf1_ref[d:2 * d],
                      preferred_element_type=jnp.float32)
        out_ref[...] = (
            jnp.dot(ohd, edw, preferred_element_type=jnp.float32)
            + jnp.dot(ohl, elw, preferred_element_type=jnp.float32)
            + bf1_ref[...])

    return pl.pallas_call(
        body, out_shape=jax.ShapeDtypeStruct((n, 128), jnp.float32))


def _make_layer1(n, npad):
    def body(pos_ref, agg_ref, w1_ref, b1_ref, g1_ref, be1_ref, eps_ref,
             x1_ref):
        u = ((1.0 + eps_ref[0, 0]) * pos_ref[:n, :]
             + agg_ref[0] + agg_ref[1])
        w1p = jnp.concatenate(
            [w1_ref[...], jnp.zeros((13, 128), jnp.float32)], axis=0)
        y = jnp.dot(u, w1p, preferred_element_type=jnp.float32)
        y = y + b1_ref[...]
        x1_ref[:n, :] = _lrelu(_bn(y, g1_ref[...], be1_ref[...]))
        x1_ref[n:, :] = jnp.zeros((npad - n, 128), jnp.float32)

    return pl.pallas_call(
        body, out_shape=jax.ShapeDtypeStruct((npad, 128), jnp.float32))


def _make_tail(n, npad):
    def body(x1_ref, agg_ref, w2_ref, b2_ref, g2_ref, be2_ref, eps_ref,
             ctrb_ref, wf1_ref, gf_ref, bef_ref, wf2t_ref, bf2_ref, out_ref):
        u = ((1.0 + eps_ref[0, 0]) * x1_ref[:n, :]
             + agg_ref[0] + agg_ref[1])
        y = jnp.dot(u, w2_ref[...], preferred_element_type=jnp.float32)
        y = y + b2_ref[...]
        x2 = _lrelu(_bn(y, g2_ref[...], be2_ref[...]))
        h = ctrb_ref[...] + jnp.dot(x2, wf1_ref[256:384],
                                    preferred_element_type=jnp.float32)
        h = _lrelu(_bn(h, gf_ref[...], bef_ref[...]))
        o = jnp.sum(h * wf2t_ref[...], axis=1, keepdims=True) + bf2_ref[0, 0]
        out_ref[...] = 1.0 / (1.0 + jnp.exp(-o))

    return pl.pallas_call(
        body, out_shape=jax.ShapeDtypeStruct((n, 1), jnp.float32))


def kernel(node_deg, node_lab, pos, edge_index, embed_deg, embed_lab,
           eps1, W1, b1, g1, be1, eps2, W2, b2, g2, be2,
           Wf1, bf1, gf, bef, Wf2, bf2):
    n = pos.shape[0]
    e = edge_index.shape[1]
    d = embed_deg.shape[1]
    npad = n + _PAD
    ept = e // _NT
    eppt = -(-ept // _CH) * _CH
    nchunk = eppt // _CH
    nsent = eppt - ept

    w = jnp.arange(_NT, dtype=jnp.int32)[:, None]
    k = jnp.arange(nsent, dtype=jnp.int32)[None, :]
    per = _PAD // _NS
    sent_src = n + (w % _NS) * per + (k % per)
    sent_dst = jnp.broadcast_to(k % n, (_NT, nsent)).astype(jnp.int32)
    er = edge_index.astype(jnp.int32).reshape(2, _NT, ept)
    src_p = jnp.concatenate([er[0], sent_src], axis=1).reshape(-1)
    dst_p = jnp.concatenate([er[1], sent_dst], axis=1).reshape(-1)

    pos_p = jnp.zeros((npad, 16), jnp.float32).at[:n, :3].set(pos)
    idx2 = jnp.stack([node_deg.astype(jnp.int32),
                      node_lab.astype(jnp.int32)], axis=1)
    row = lambda a: a.reshape(1, -1).astype(jnp.float32)
    sca = lambda a: a.reshape(1, 1).astype(jnp.float32)

    ctrb = _make_ctrb(n)(idx2, embed_deg, embed_lab, Wf1, row(bf1))
    agg1 = _make_edge_agg(n, npad, nchunk, 16, True, 4, 4)(
        pos_p, src_p, dst_p)
    x1 = _make_layer1(n, npad)(pos_p, agg1, W1, row(b1), row(g1), row(be1),
                               sca(eps1))
    agg2 = _make_edge_agg(n, npad, nchunk, d, False, 3, 4)(
        x1, src_p, dst_p, jnp.zeros((n, d), jnp.float32))
    out = _make_tail(n, npad)(x1, agg2, W2, row(b2), row(g2), row(be2),
                              sca(eps2), ctrb, Wf1, row(gf), row(bef),
                              row(Wf2.T), sca(bf2))
    return out[:, 0]


# speedup vs baseline: 1.0857x; 1.0857x over previous
import functools

import jax
import jax.numpy as jnp
from jax import lax
from jax.experimental import pallas as pl
from jax.experimental.pallas import tpu as pltpu
from jax.experimental.pallas import tpu_sc as plsc

_NC = 2
_NS = 16
_NT = _NC * _NS
_CH = 128


def _make_edge_agg(n, ept, d, stage_x, nb_rows, nb_idx):
    mesh = plsc.VectorSubcoreMesh(core_axis_name="c", subcore_axis_name="s")
    nblk = n // 80
    nfull = ept // _CH
    tail = ept % _CH

    scratch = (
        [pltpu.VMEM((_CH,), jnp.int32) for _ in range(nb_idx)]
        + [pltpu.VMEM((_CH,), jnp.int32) for _ in range(nb_idx)]
        + [pltpu.VMEM((_CH, d), jnp.float32) for _ in range(nb_rows)]
        + [pltpu.VMEM_SHARED((n, d), jnp.float32)]
        + ([pltpu.VMEM_SHARED((n, d), jnp.float32)] if stage_x else [])
        + [pltpu.SemaphoreType.DMA] * (2 * nb_idx + 2 * nb_rows)
    )

    @functools.partial(
        pl.kernel,
        out_type=jax.ShapeDtypeStruct((_NC, n, d), jnp.float32),
        mesh=mesh,
        compiler_params=pltpu.CompilerParams(use_tc_tiling_on_sc=False),
        scratch_types=scratch,
    )
    def agg(x_hbm, src_hbm, dst_hbm, out_hbm, *refs):
        sidx = refs[0:nb_idx]
        didx = refs[nb_idx:2 * nb_idx]
        rows = refs[2 * nb_idx:2 * nb_idx + nb_rows]
        agg_sh = refs[2 * nb_idx + nb_rows]
        x_src = refs[2 * nb_idx + nb_rows + 1] if stage_x else x_hbm
        sems = refs[2 * nb_idx + nb_rows + 1 + (1 if stage_x else 0):]
        ssem = sems[0:nb_idx]
        dsem = sems[nb_idx:2 * nb_idx]
        gsem = sems[2 * nb_idx:2 * nb_idx + nb_rows]
        wsem = sems[2 * nb_idx + nb_rows:]

        c = lax.axis_index("c")
        s = lax.axis_index("s")
        wid = c * _NS + s
        base = pl.multiple_of(wid * ept, 8)

        def iload(i):
            b = i % nb_idx
            sl = pl.ds(base + i * _CH, _CH)
            return (pltpu.async_copy(src_hbm.at[sl], sidx[b], ssem[b]),
                    pltpu.async_copy(dst_hbm.at[sl], didx[b], dsem[b]))

        def gather(i):
            b = i % nb_rows
            return pltpu.async_copy(x_src.at[sidx[i % nb_idx]], rows[b],
                                    gsem[b])

        def scat(i):
            b = i % nb_rows
            return pltpu.async_copy(rows[b], agg_sh.at[didx[i % nb_idx]],
                                    wsem[b], add=True)

        idd = {i: iload(i) for i in range(min(3, nfull))}

        def zrow(r, carry):
            for cc in range(d // 16):
                rows[0][r, pl.ds(cc * 16, 16)] = jnp.zeros((16,), jnp.float32)
            return carry
        lax.fori_loop(0, 80, zrow, 0)
        for j in range(-(-nblk // _NS)):
            blk = j * _NS + s

            @pl.when(blk < nblk)
            def _():
                pltpu.sync_copy(rows[0].at[pl.ds(0, 80)],
                                agg_sh.at[pl.ds(blk * 80, 80)])

        if stage_x:
            @pl.when(s == 0)
            def _():
                pltpu.sync_copy(x_hbm, x_src)

        plsc.subcore_barrier()

        gd = {}
        for i in range(min(2, nfull)):
            idd[i][0].wait()
            gd[i] = gather(i)

        sd = {}
        for i in range(nfull):
            gd[i].wait()
            idd[i][1].wait()
            sd[i] = scat(i)
            if i + 3 < nfull:
                if i >= 1:
                    sd[i - 1].wait()
                idd[i + 3] = iload(i + 3)
            if i + 2 < nfull:
                idd[i + 2][0].wait()
                gd[i + 2] = gather(i + 2)
        for i in range(max(0, nfull - 4), nfull):
            sd[i].wait()

        if tail:
            tsl = pl.ds(base + nfull * _CH, tail)
            vsl = pl.ds(0, tail)
            pltpu.sync_copy(src_hbm.at[tsl], sidx[0].at[vsl])
            pltpu.sync_copy(dst_hbm.at[tsl], didx[0].at[vsl])
            pltpu.async_copy(x_src.at[sidx[0].at[vsl]], rows[0].at[vsl],
                             gsem[0]).wait()
            pltpu.async_copy(rows[0].at[vsl], agg_sh.at[didx[0].at[vsl]],
                             wsem[0], add=True).wait()

        plsc.subcore_barrier()

        for cc in range(_NC):
            @pl.when((s == 0) & (c == cc))
            def _():
                pltpu.sync_copy(agg_sh, out_hbm.at[cc])

    return agg


def _lrelu(x):
    return jnp.where(x >= 0, x, 0.01 * x)


def _bn(y, g, b):
    m = jnp.mean(y, axis=0, keepdims=True)
    v = jnp.mean((y - m) * (y - m), axis=0, keepdims=True)
    return g * (y - m) * lax.rsqrt(v + 1e-5) + b


def _make_ctrb(n):
    def body(idx_ref, ed_ref, el_ref, wf1_ref, bf1_ref, out_ref):
        d = 128
        it = lax.broadcasted_iota(jnp.int32, (n, 128), 1)
        ohd = (idx_ref[:, 0:1] == it).astype(jnp.float32)
        ohl = (idx_ref[:, 1:2] == it).astype(jnp.float32)
        ed_p = jnp.concatenate(
            [ed_ref[...], jnp.zeros((128 - ed_ref.shape[0], d), jnp.float32)],
            axis=0)
        el_p = jnp.concatenate(
            [el_ref[...], jnp.zeros((128 - el_ref.shape[0], d), jnp.float32)],
            axis=0)
        edw = jnp.dot(ed_p, wf1_ref[0:d], preferred_element_type=jnp.float32)
        elw = jnp.dot(el_p, wf1_ref[d:2 * d],
                      preferred_element_type=jnp.float32)
        out_ref[...] = (
            jnp.dot(ohd, edw, preferred_element_type=jnp.float32)
            + jnp.dot(ohl, elw, preferred_element_type=jnp.float32)
            + bf1_ref[...])

    return pl.pallas_call(
        body, out_shape=jax.ShapeDtypeStruct((n, 128), jnp.float32))


def _make_layer1(n):
    def body(pos_ref, agg_ref, w1_ref, b1_ref, g1_ref, be1_ref, eps_ref,
             x1_ref):
        u = ((1.0 + eps_ref[0, 0]) * pos_ref[...]
             + agg_ref[0] + agg_ref[1])
        w1p = jnp.concatenate(
            [w1_ref[...], jnp.zeros((13, 128), jnp.float32)], axis=0)
        y = jnp.dot(u, w1p, preferred_element_type=jnp.float32)
        y = y + b1_ref[...]
        x1_ref[...] = _lrelu(_bn(y, g1_ref[...], be1_ref[...]))

    return pl.pallas_call(
        body, out_shape=jax.ShapeDtypeStruct((n, 128), jnp.float32))


def _make_tail(n):
    def body(x1_ref, agg_ref, w2_ref, b2_ref, g2_ref, be2_ref, eps_ref,
             ctrb_ref, wf1_ref, gf_ref, bef_ref, wf2t_ref, bf2_ref, out_ref):
        u = ((1.0 + eps_ref[0, 0]) * x1_ref[...]
             + agg_ref[0] + agg_ref[1])
        y = jnp.dot(u, w2_ref[...], preferred_element_type=jnp.float32)
        y = y + b2_ref[...]
        x2 = _lrelu(_bn(y, g2_ref[...], be2_ref[...]))
        h = ctrb_ref[...] + jnp.dot(x2, wf1_ref[256:384],
                                    preferred_element_type=jnp.float32)
        h = _lrelu(_bn(h, gf_ref[...], bef_ref[...]))
        o = jnp.sum(h * wf2t_ref[...], axis=1, keepdims=True) + bf2_ref[0, 0]
        out_ref[...] = 1.0 / (1.0 + jnp.exp(-o))

    return pl.pallas_call(
        body, out_shape=jax.ShapeDtypeStruct((n, 1), jnp.float32))


def kernel(node_deg, node_lab, pos, edge_index, embed_deg, embed_lab,
           eps1, W1, b1, g1, be1, eps2, W2, b2, g2, be2,
           Wf1, bf1, gf, bef, Wf2, bf2):
    n = pos.shape[0]
    e = edge_index.shape[1]
    d = embed_deg.shape[1]
    ept = e // _NT

    src_p = edge_index[0].astype(jnp.int32)
    dst_p = edge_index[1].astype(jnp.int32)
    pos_p = jnp.zeros((n, 16), jnp.float32).at[:, :3].set(pos)
    idx2 = jnp.stack([node_deg.astype(jnp.int32),
                      node_lab.astype(jnp.int32)], axis=1)
    row = lambda a: a.reshape(1, -1).astype(jnp.float32)
    sca = lambda a: a.reshape(1, 1).astype(jnp.float32)

    ctrb = _make_ctrb(n)(idx2, embed_deg, embed_lab, Wf1, row(bf1))
    agg1 = _make_edge_agg(n, ept, 16, True, 4, 4)(pos_p, src_p, dst_p)
    x1 = _make_layer1(n)(pos_p, agg1, W1, row(b1), row(g1), row(be1),
                         sca(eps1))
    agg2 = _make_edge_agg(n, ept, d, False, 3, 4)(x1, src_p, dst_p)
    out = _make_tail(n)(x1, agg2, W2, row(b2), row(g2), row(be2), sca(eps2),
                        ctrb, Wf1, row(gf), row(bef), row(Wf2.T), sca(bf2))
    return out[:, 0]


# speedup vs baseline: 1.1004x; 1.0135x over previous
import functools

import jax
import jax.numpy as jnp
from jax import lax
from jax.experimental import pallas as pl
from jax.experimental.pallas import tpu as pltpu
from jax.experimental.pallas import tpu_sc as plsc

_NC = 2
_NS = 16
_NT = _NC * _NS
_CH = 128


def _make_edge_agg(n, ept, d, stage_x, nb_rows, nb_idx):
    mesh = plsc.VectorSubcoreMesh(core_axis_name="c", subcore_axis_name="s")
    nblk = n // 80
    nfull = ept // _CH
    tail = ept % _CH

    scratch = (
        [pltpu.VMEM((_CH,), jnp.int32) for _ in range(nb_idx)]
        + [pltpu.VMEM((_CH,), jnp.int32) for _ in range(nb_idx)]
        + [pltpu.VMEM((_CH, d), jnp.float32) for _ in range(nb_rows)]
        + [pltpu.VMEM_SHARED((n, d), jnp.float32)]
        + ([pltpu.VMEM_SHARED((n, d), jnp.float32)] if stage_x else [])
        + [pltpu.SemaphoreType.DMA] * (2 * nb_idx + 2 * nb_rows)
    )

    @functools.partial(
        pl.kernel,
        out_type=jax.ShapeDtypeStruct((_NC, n, d), jnp.float32),
        mesh=mesh,
        compiler_params=pltpu.CompilerParams(use_tc_tiling_on_sc=False),
        scratch_types=scratch,
    )
    def agg(x_hbm, src_hbm, dst_hbm, out_hbm, *refs):
        sidx = refs[0:nb_idx]
        didx = refs[nb_idx:2 * nb_idx]
        rows = refs[2 * nb_idx:2 * nb_idx + nb_rows]
        agg_sh = refs[2 * nb_idx + nb_rows]
        x_src = refs[2 * nb_idx + nb_rows + 1] if stage_x else x_hbm
        sems = refs[2 * nb_idx + nb_rows + 1 + (1 if stage_x else 0):]
        ssem = sems[0:nb_idx]
        dsem = sems[nb_idx:2 * nb_idx]
        gsem = sems[2 * nb_idx:2 * nb_idx + nb_rows]
        wsem = sems[2 * nb_idx + nb_rows:]

        c = lax.axis_index("c")
        s = lax.axis_index("s")
        wid = c * _NS + s

        def iload(i):
            b = i % nb_idx
            sl = pl.ds(i * _CH, _CH)
            return (pltpu.async_copy(src_hbm.at[wid, sl], sidx[b], ssem[b]),
                    pltpu.async_copy(dst_hbm.at[wid, sl], didx[b], dsem[b]))

        def gather(i):
            b = i % nb_rows
            return pltpu.async_copy(x_src.at[sidx[i % nb_idx]], rows[b],
                                    gsem[b])

        def scat(i):
            b = i % nb_rows
            return pltpu.async_copy(rows[b], agg_sh.at[didx[i % nb_idx]],
                                    wsem[b], add=True)

        idd = {i: iload(i) for i in range(min(3, nfull))}

        def zrow(r, carry):
            for cc in range(d // 16):
                rows[0][r, pl.ds(cc * 16, 16)] = jnp.zeros((16,), jnp.float32)
            return carry
        lax.fori_loop(0, 80, zrow, 0)
        for j in range(-(-nblk // _NS)):
            blk = j * _NS + s

            @pl.when(blk < nblk)
            def _():
                pltpu.sync_copy(rows[0].at[pl.ds(0, 80)],
                                agg_sh.at[pl.ds(blk * 80, 80)])

        if stage_x:
            @pl.when(s == 0)
            def _():
                pltpu.sync_copy(x_hbm, x_src)

        plsc.subcore_barrier()

        gd = {}
        for i in range(min(2, nfull)):
            idd[i][0].wait()
            gd[i] = gather(i)

        sd = {}
        for i in range(nfull):
            gd[i].wait()
            idd[i][1].wait()
            sd[i] = scat(i)
            if i + 3 < nfull:
                if i >= 1:
                    sd[i - 1].wait()
                idd[i + 3] = iload(i + 3)
            if i + 2 < nfull:
                idd[i + 2][0].wait()
                gd[i + 2] = gather(i + 2)
        for i in range(max(0, nfull - 4), nfull):
            sd[i].wait()

        if tail:
            tsl = pl.ds(nfull * _CH, tail)
            vsl = pl.ds(0, tail)
            pltpu.sync_copy(src_hbm.at[wid, tsl], sidx[0].at[vsl])
            pltpu.sync_copy(dst_hbm.at[wid, tsl], didx[0].at[vsl])
            pltpu.async_copy(x_src.at[sidx[0].at[vsl]], rows[0].at[vsl],
                             gsem[0]).wait()
            pltpu.async_copy(rows[0].at[vsl], agg_sh.at[didx[0].at[vsl]],
                             wsem[0], add=True).wait()

        plsc.subcore_barrier()

        for cc in range(_NC):
            @pl.when((s == 0) & (c == cc))
            def _():
                pltpu.sync_copy(agg_sh, out_hbm.at[cc])

    return agg


def _lrelu(x):
    return jnp.where(x >= 0, x, 0.01 * x)


def _bn(y, g, b):
    m = jnp.mean(y, axis=0, keepdims=True)
    v = jnp.mean((y - m) * (y - m), axis=0, keepdims=True)
    return g * (y - m) * lax.rsqrt(v + 1e-5) + b


def _make_ctrb(n):
    def body(idx_ref, ed_ref, el_ref, wf1_ref, bf1_ref, out_ref):
        d = 128
        it = lax.broadcasted_iota(jnp.int32, (n, 128), 1)
        ohd = (idx_ref[:, 0:1] == it).astype(jnp.float32)
        ohl = (idx_ref[:, 1:2] == it).astype(jnp.float32)
        ed_p = jnp.concatenate(
            [ed_ref[...], jnp.zeros((128 - ed_ref.shape[0], d), jnp.float32)],
            axis=0)
        el_p = jnp.concatenate(
            [el_ref[...], jnp.zeros((128 - el_ref.shape[0], d), jnp.float32)],
            axis=0)
        edw = jnp.dot(ed_p, wf1_ref[0:d], preferred_element_type=jnp.float32)
        elw = jnp.dot(el_p, wf1_ref[d:2 * d],
                      preferred_element_type=jnp.float32)
        out_ref[...] = (
            jnp.dot(ohd, edw, preferred_element_type=jnp.float32)
            + jnp.dot(ohl, elw, preferred_element_type=jnp.float32)
            + bf1_ref[...])

    return pl.pallas_call(
        body, out_shape=jax.ShapeDtypeStruct((n, 128), jnp.float32))


def _make_layer1(n):
    def body(pos_ref, agg_ref, w1_ref, b1_ref, g1_ref, be1_ref, eps_ref,
             x1_ref):
        u = ((1.0 + eps_ref[0, 0]) * pos_ref[...]
             + agg_ref[0] + agg_ref[1])
        w1p = jnp.concatenate(
            [w1_ref[...], jnp.zeros((13, 128), jnp.float32)], axis=0)
        y = jnp.dot(u, w1p, preferred_element_type=jnp.float32)
        y = y + b1_ref[...]
        x1_ref[...] = _lrelu(_bn(y, g1_ref[...], be1_ref[...]))

    return pl.pallas_call(
        body, out_shape=jax.ShapeDtypeStruct((n, 128), jnp.float32))


def _make_tail(n):
    def body(x1_ref, agg_ref, w2_ref, b2_ref, g2_ref, be2_ref, eps_ref,
             ctrb_ref, wf1_ref, gf_ref, bef_ref, wf2t_ref, bf2_ref, out_ref):
        u = ((1.0 + eps_ref[0, 0]) * x1_ref[...]
             + agg_ref[0] + agg_ref[1])
        y = jnp.dot(u, w2_ref[...], preferred_element_type=jnp.float32)
        y = y + b2_ref[...]
        x2 = _lrelu(_bn(y, g2_ref[...], be2_ref[...]))
        h = ctrb_ref[...] + jnp.dot(x2, wf1_ref[256:384],
                                    preferred_element_type=jnp.float32)
        h = _lrelu(_bn(h, gf_ref[...], bef_ref[...]))
        o = jnp.sum(h * wf2t_ref[...], axis=1) + bf2_ref[0, 0]
        out_ref[...] = 1.0 / (1.0 + jnp.exp(-o))

    return pl.pallas_call(
        body, out_shape=jax.ShapeDtypeStruct((n,), jnp.float32))


def kernel(node_deg, node_lab, pos, edge_index, embed_deg, embed_lab,
           eps1, W1, b1, g1, be1, eps2, W2, b2, g2, be2,
           Wf1, bf1, gf, bef, Wf2, bf2):
    n = pos.shape[0]
    e = edge_index.shape[1]
    d = embed_deg.shape[1]
    ept = e // _NT

    src_p = edge_index[0].astype(jnp.int32).reshape(_NT, ept)
    dst_p = edge_index[1].astype(jnp.int32).reshape(_NT, ept)
    pos_p = jnp.zeros((n, 16), jnp.float32).at[:, :3].set(pos)
    idx2 = jnp.stack([node_deg.astype(jnp.int32),
                      node_lab.astype(jnp.int32)], axis=1)
    row = lambda a: a.reshape(1, -1).astype(jnp.float32)
    sca = lambda a: a.reshape(1, 1).astype(jnp.float32)

    ctrb = _make_ctrb(n)(idx2, embed_deg, embed_lab, Wf1, row(bf1))
    agg1 = _make_edge_agg(n, ept, 16, True, 4, 4)(pos_p, src_p, dst_p)
    x1 = _make_layer1(n)(pos_p, agg1, W1, row(b1), row(g1), row(be1),
                         sca(eps1))
    agg2 = _make_edge_agg(n, ept, d, False, 3, 4)(x1, src_p, dst_p)
    out = _make_tail(n)(x1, agg2, W2, row(b2), row(g2), row(be2), sca(eps2),
                        ctrb, Wf1, row(gf), row(bef), row(Wf2.T), sca(bf2))
    return out
